# baseline (speedup 1.0000x reference)
import functools

import jax
import jax.numpy as jnp
from jax import lax
from jax.experimental import pallas as pl
from jax.experimental.pallas import tpu as pltpu
from jax.experimental.pallas import tpu_sc as plsc

N_NODES = 10000
N_EDGES = 320000
D = 128

NC = 2
NS = 16
NW = NC * NS
EPW = N_EDGES // NW
CH = 80
NCHUNK = EPW // CH
G = 25
NGRP = NCHUNK // G
ROWS_PT = 624
ROWS_TAIL = N_NODES - NS * ROWS_PT

_MESH = plsc.VectorSubcoreMesh(core_axis_name="c", subcore_axis_name="s")


def _zero_shared(zeros_hbm, sh, s):
    r0 = s * ROWS_PT
    pltpu.sync_copy(zeros_hbm.at[pl.ds(r0, ROWS_PT)],
                    sh.at[pl.ds(r0, ROWS_PT)])

    @pl.when(s == 0)
    def _():
        t0 = NS * ROWS_PT
        pltpu.sync_copy(zeros_hbm.at[pl.ds(t0, ROWS_TAIL)],
                        sh.at[pl.ds(t0, ROWS_TAIL)])


def _writeback_shared(sh, out, c, s):
    r0 = s * ROWS_PT
    pltpu.sync_copy(sh.at[pl.ds(r0, ROWS_PT)],
                    out.at[c, pl.ds(r0, ROWS_PT)])

    @pl.when(s == 0)
    def _():
        t0 = NS * ROWS_PT
        pltpu.sync_copy(sh.at[pl.ds(t0, ROWS_TAIL)],
                        out.at[c, pl.ds(t0, ROWS_TAIL)])


def _seg_body(x_hbm, src_hbm, dst_hbm, zeros_hbm, agg_out,
              src_v, dst_v, rows_a, rows_b, agg_sh, gsa, gsb, ssa, ssb):
    c = lax.axis_index("c")
    s = lax.axis_index("s")
    wid = c * NS + s

    _zero_shared(zeros_hbm, agg_sh, s)
    plsc.subcore_barrier()

    def gather(j, buf, sem):
        pltpu.async_copy(x_hbm.at[src_v.at[j]], buf, sem)

    def wait_gather(j, buf, sem):
        pltpu.make_async_copy(x_hbm.at[src_v.at[j]], buf, sem).wait()

    def scatter(j, buf, sem):
        pltpu.async_copy(buf, agg_sh.at[dst_v.at[j]], sem, add=True)

    def wait_scatter(j, buf, sem):
        pltpu.make_async_copy(buf, agg_sh.at[dst_v.at[j]], sem).wait()

    def pair_body(k, carry):
        j0 = 2 * k
        wait_gather(j0, rows_a, gsa)
        scatter(j0, rows_a, ssa)

        @pl.when(k > 0)
        def _():
            wait_scatter(j0 - 1, rows_b, ssb)

        gather(j0 + 1, rows_b, gsb)
        wait_gather(j0 + 1, rows_b, gsb)
        scatter(j0 + 1, rows_b, ssb)
        wait_scatter(j0, rows_a, ssa)
        gather(j0 + 2, rows_a, gsa)
        return carry

    def group_body(g, carry):
        pltpu.sync_copy(src_hbm.at[wid, g], src_v)
        pltpu.sync_copy(dst_hbm.at[wid, g], dst_v)
        gather(0, rows_a, gsa)
        carry = lax.fori_loop(0, (G - 1) // 2, pair_body, carry)
        wait_gather(G - 1, rows_a, gsa)
        scatter(G - 1, rows_a, ssa)
        wait_scatter(G - 2, rows_b, ssb)
        wait_scatter(G - 1, rows_a, ssa)
        return carry

    lax.fori_loop(0, NGRP, group_body, 0)
    plsc.subcore_barrier()
    _writeback_shared(agg_sh, agg_out, c, s)


NP = N_NODES + 16
NH = 8


def _deg_body(dst_hbm, deg_out, dst_v, hist_v):
    c = lax.axis_index("c")
    s = lax.axis_index("s")
    wid = c * NS + s

    def zero(r, carry):
        hist_v[pl.ds(16 * r, 16)] = jnp.zeros((16,), jnp.float32)
        return carry

    lax.fori_loop(0, NH * NP // 16, zero, 0)
    pltpu.sync_copy(dst_hbm.at[pl.ds(wid * EPW, EPW)], dst_v)

    lane = lax.iota(jnp.int32, 16)
    base = (lane & (NH - 1)) * NP
    mask_lo = lane < NH
    mask_hi = lane >= NH
    ones = jnp.ones((16,), jnp.float32)

    def count(e, carry):
        idxv = dst_v[pl.ds(16 * e, 16)] + base
        plsc.addupdate_scatter(hist_v, [idxv], ones, mask=mask_lo)
        plsc.addupdate_scatter(hist_v, [idxv], ones, mask=mask_hi)
        return carry

    lax.fori_loop(0, EPW // 16, count, 0)

    def merge(r, carry):
        acc = hist_v[pl.ds(16 * r, 16)]
        for k in range(1, NH):
            acc = acc + hist_v[pl.ds(k * NP + 16 * r, 16)]
        hist_v[pl.ds(16 * r, 16)] = acc
        return carry

    lax.fori_loop(0, N_NODES // 16, merge, 0)
    pltpu.sync_copy(hist_v.at[pl.ds(0, N_NODES)],
                    deg_out.at[pl.ds(wid * N_NODES, N_NODES)])


_seg_sum = pl.kernel(
    _seg_body,
    out_type=jax.ShapeDtypeStruct((NC, N_NODES, D), jnp.float32),
    mesh=_MESH,
    scratch_types=[
        pltpu.VMEM((G, CH), jnp.int32),
        pltpu.VMEM((G, CH), jnp.int32),
        pltpu.VMEM((CH, D), jnp.float32),
        pltpu.VMEM((CH, D), jnp.float32),
        pltpu.VMEM_SHARED((N_NODES, D), jnp.float32),
        pltpu.SemaphoreType.DMA,
        pltpu.SemaphoreType.DMA,
        pltpu.SemaphoreType.DMA,
        pltpu.SemaphoreType.DMA,
    ],
)
assert G % 2 == 1

_deg_sum = pl.kernel(
    _deg_body,
    out_type=jax.ShapeDtypeStruct((NW * N_NODES,), jnp.float32),
    mesh=_MESH,
    scratch_types=[
        pltpu.VMEM((EPW,), jnp.int32),
        pltpu.VMEM((NH * NP,), jnp.float32),
    ],
    compiler_params=pltpu.CompilerParams(needs_layout_passes=False),
)


def _layer_body(relu, p_ref, degp_ref, x_ref, wl_ref, wr_ref, b_ref, out_ref):
    agg = p_ref[0] + p_ref[1]
    deg = jnp.sum(degp_ref[...], axis=1, keepdims=True)
    inv = 1.0 / jnp.maximum(deg, 1.0)
    hp = jax.lax.Precision.HIGHEST
    acc = (jnp.dot(agg * inv, wl_ref[...], precision=hp,
                   preferred_element_type=jnp.float32)
           + jnp.dot(x_ref[...], wr_ref[...], precision=hp,
                     preferred_element_type=jnp.float32)
           + b_ref[...])
    out_ref[...] = jnp.maximum(acc, 0.0) if relu else acc


def _tc_layer(p, degp, x, Wl, Wr, b, relu):
    BR = 5000
    grid = (N_NODES // BR,)
    return pl.pallas_call(
        functools.partial(_layer_body, relu),
        grid=grid,
        in_specs=[
            pl.BlockSpec((NC, BR, D), lambda i: (0, i, 0)),
            pl.BlockSpec((BR, NW), lambda i: (i, 0)),
            pl.BlockSpec((BR, D), lambda i: (i, 0)),
            pl.BlockSpec((D, D), lambda i: (0, 0)),
            pl.BlockSpec((D, D), lambda i: (0, 0)),
            pl.BlockSpec((1, D), lambda i: (0, 0)),
        ],
        out_specs=pl.BlockSpec((BR, D), lambda i: (i, 0)),
        out_shape=jax.ShapeDtypeStruct((N_NODES, D), jnp.float32),
    )(p, degp, x, Wl, Wr, b.reshape(1, D))


def kernel(x, edge_index, Wl1, Wr1, b1, Wl2, Wr2, b2):
    src = edge_index[0].astype(jnp.int32).reshape(NW, NGRP, G, CH)
    dst = edge_index[1].astype(jnp.int32).reshape(NW, NGRP, G, CH)
    dst_flat = edge_index[1].astype(jnp.int32)
    zeros = jnp.zeros((N_NODES, D), jnp.float32)

    degp = jnp.transpose(_deg_sum(dst_flat).reshape(NW, N_NODES))
    p1 = _seg_sum(x, src, dst, zeros)
    h = _tc_layer(p1, degp, x, Wl1, Wr1, b1, relu=True)
    p2 = _seg_sum(h, src, dst, zeros)
    out = _tc_layer(p2, degp, h, Wl2, Wr2, b2, relu=False)
    return out


# speedup vs baseline: 1.0345x; 1.0345x over previous
import functools

import jax
import jax.numpy as jnp
from jax import lax
from jax.experimental import pallas as pl
from jax.experimental.pallas import tpu as pltpu
from jax.experimental.pallas import tpu_sc as plsc

N_NODES = 10000
N_EDGES = 320000
D = 128

NC = 2
NS = 16
NW = NC * NS
EPW = N_EDGES // NW
CH = 80
NCHUNK = EPW // CH
G = 25
NGRP = NCHUNK // G
ROWS_PT = 624
ROWS_TAIL = N_NODES - NS * ROWS_PT

_MESH = plsc.VectorSubcoreMesh(core_axis_name="c", subcore_axis_name="s")


def _zero_shared(zeros_hbm, sh, s):
    r0 = s * ROWS_PT
    pltpu.sync_copy(zeros_hbm.at[pl.ds(r0, ROWS_PT)],
                    sh.at[pl.ds(r0, ROWS_PT)])

    @pl.when(s == 0)
    def _():
        t0 = NS * ROWS_PT
        pltpu.sync_copy(zeros_hbm.at[pl.ds(t0, ROWS_TAIL)],
                        sh.at[pl.ds(t0, ROWS_TAIL)])


def _writeback_shared(sh, out, c, s):
    r0 = s * ROWS_PT
    pltpu.sync_copy(sh.at[pl.ds(r0, ROWS_PT)],
                    out.at[c, pl.ds(r0, ROWS_PT)])

    @pl.when(s == 0)
    def _():
        t0 = NS * ROWS_PT
        pltpu.sync_copy(sh.at[pl.ds(t0, ROWS_TAIL)],
                        out.at[c, pl.ds(t0, ROWS_TAIL)])


def _seg_body(x_hbm, src_hbm, dst_hbm, zeros_hbm, agg_out,
              src_v, dst_v, rows_a, rows_b, agg_sh, gsa, gsb, ssa, ssb):
    c = lax.axis_index("c")
    s = lax.axis_index("s")
    wid = c * NS + s

    _zero_shared(zeros_hbm, agg_sh, s)
    plsc.subcore_barrier()

    def gather(j, buf, sem):
        pltpu.async_copy(x_hbm.at[src_v.at[j]], buf, sem)

    def wait_gather(j, buf, sem):
        pltpu.make_async_copy(x_hbm.at[src_v.at[j]], buf, sem).wait()

    def scatter(j, buf, sem):
        pltpu.async_copy(buf, agg_sh.at[dst_v.at[j]], sem, add=True)

    def wait_scatter(j, buf, sem):
        pltpu.make_async_copy(buf, agg_sh.at[dst_v.at[j]], sem).wait()

    def pair_body(k, carry):
        j0 = 2 * k
        wait_gather(j0, rows_a, gsa)
        scatter(j0, rows_a, ssa)

        @pl.when(k > 0)
        def _():
            wait_scatter(j0 - 1, rows_b, ssb)

        gather(j0 + 1, rows_b, gsb)
        wait_gather(j0 + 1, rows_b, gsb)
        scatter(j0 + 1, rows_b, ssb)
        wait_scatter(j0, rows_a, ssa)
        gather(j0 + 2, rows_a, gsa)
        return carry

    def group_body(g, carry):
        pltpu.sync_copy(src_hbm.at[wid, g], src_v)
        pltpu.sync_copy(dst_hbm.at[wid, g], dst_v)
        gather(0, rows_a, gsa)
        carry = lax.fori_loop(0, (G - 1) // 2, pair_body, carry)
        wait_gather(G - 1, rows_a, gsa)
        scatter(G - 1, rows_a, ssa)
        wait_scatter(G - 2, rows_b, ssb)
        wait_scatter(G - 1, rows_a, ssa)
        return carry

    lax.fori_loop(0, NGRP, group_body, 0)
    plsc.subcore_barrier()
    _writeback_shared(agg_sh, agg_out, c, s)


NP = N_NODES + 16
NH = 8


def _deg_body(dst_hbm, deg_out, dst_v, hist_v):
    c = lax.axis_index("c")
    s = lax.axis_index("s")
    wid = c * NS + s

    def zero(r, carry):
        hist_v[pl.ds(16 * r, 16)] = jnp.zeros((16,), jnp.float32)
        return carry

    lax.fori_loop(0, NH * NP // 16, zero, 0)
    pltpu.sync_copy(dst_hbm.at[pl.ds(wid * EPW, EPW)], dst_v)

    lane = lax.iota(jnp.int32, 16)
    base = (lane & (NH - 1)) * NP
    mask_lo = lane < NH
    mask_hi = lane >= NH
    ones = jnp.ones((16,), jnp.float32)

    def count(e, carry):
        idxv = dst_v[pl.ds(16 * e, 16)] + base
        plsc.addupdate_scatter(hist_v, [idxv], ones, mask=mask_lo)
        plsc.addupdate_scatter(hist_v, [idxv], ones, mask=mask_hi)
        return carry

    lax.fori_loop(0, EPW // 16, count, 0)

    def merge(r, carry):
        acc = hist_v[pl.ds(16 * r, 16)]
        for k in range(1, NH):
            acc = acc + hist_v[pl.ds(k * NP + 16 * r, 16)]
        hist_v[pl.ds(16 * r, 16)] = acc
        return carry

    lax.fori_loop(0, N_NODES // 16, merge, 0)
    pltpu.sync_copy(hist_v.at[pl.ds(0, N_NODES)],
                    deg_out.at[pl.ds(wid * N_NODES, N_NODES)])


_seg_sum = pl.kernel(
    _seg_body,
    out_type=jax.ShapeDtypeStruct((NC, N_NODES, D), jnp.float32),
    mesh=_MESH,
    scratch_types=[
        pltpu.VMEM((G, CH), jnp.int32),
        pltpu.VMEM((G, CH), jnp.int32),
        pltpu.VMEM((CH, D), jnp.float32),
        pltpu.VMEM((CH, D), jnp.float32),
        pltpu.VMEM_SHARED((N_NODES, D), jnp.float32),
        pltpu.SemaphoreType.DMA,
        pltpu.SemaphoreType.DMA,
        pltpu.SemaphoreType.DMA,
        pltpu.SemaphoreType.DMA,
    ],
)
assert G % 2 == 1

_deg_sum = pl.kernel(
    _deg_body,
    out_type=jax.ShapeDtypeStruct((NW * N_NODES,), jnp.float32),
    mesh=_MESH,
    scratch_types=[
        pltpu.VMEM((EPW,), jnp.int32),
        pltpu.VMEM((NH * NP,), jnp.float32),
    ],
    compiler_params=pltpu.CompilerParams(needs_layout_passes=False),
)


def _layer_body(relu, p_ref, degp_ref, x_ref, wl_ref, wr_ref, b_ref, out_ref):
    agg = p_ref[0] + p_ref[1]
    deg = jnp.sum(degp_ref[...], axis=1, keepdims=True)
    inv = 1.0 / jnp.maximum(deg, 1.0)
    hp = jax.lax.Precision.HIGHEST
    acc = (jnp.dot(agg * inv, wl_ref[...], precision=hp,
                   preferred_element_type=jnp.float32)
           + jnp.dot(x_ref[...], wr_ref[...], precision=hp,
                     preferred_element_type=jnp.float32)
           + b_ref[...])
    out_ref[...] = jnp.maximum(acc, 0.0) if relu else acc


def _tc_layer(p, degp, x, Wl, Wr, b, relu):
    BR = 2000
    grid = (N_NODES // BR,)
    return pl.pallas_call(
        functools.partial(_layer_body, relu),
        grid=grid,
        in_specs=[
            pl.BlockSpec((NC, BR, D), lambda i: (0, i, 0)),
            pl.BlockSpec((BR, NW), lambda i: (i, 0)),
            pl.BlockSpec((BR, D), lambda i: (i, 0)),
            pl.BlockSpec((D, D), lambda i: (0, 0)),
            pl.BlockSpec((D, D), lambda i: (0, 0)),
            pl.BlockSpec((1, D), lambda i: (0, 0)),
        ],
        out_specs=pl.BlockSpec((BR, D), lambda i: (i, 0)),
        out_shape=jax.ShapeDtypeStruct((N_NODES, D), jnp.float32),
    )(p, degp, x, Wl, Wr, b.reshape(1, D))


def kernel(x, edge_index, Wl1, Wr1, b1, Wl2, Wr2, b2):
    src = edge_index[0].astype(jnp.int32).reshape(NW, NGRP, G, CH)
    dst = edge_index[1].astype(jnp.int32).reshape(NW, NGRP, G, CH)
    dst_flat = edge_index[1].astype(jnp.int32)
    zeros = jnp.zeros((N_NODES, D), jnp.float32)

    degp = jnp.transpose(_deg_sum(dst_flat).reshape(NW, N_NODES))
    p1 = _seg_sum(x, src, dst, zeros)
    h = _tc_layer(p1, degp, x, Wl1, Wr1, b1, relu=True)
    p2 = _seg_sum(h, src, dst, zeros)
    out = _tc_layer(p2, degp, h, Wl2, Wr2, b2, relu=False)
    return out
